# Initial kernel scaffold; baseline (speedup 1.0000x reference)
import jax
import jax.numpy as jnp
from jax.experimental import pallas as pl


def kernel(z, pos, params):
    raise NotImplementedError("write your pallas kernel here")


# speedup vs baseline: 4.8477x; 4.8477x over previous
import functools

import jax
import jax.numpy as jnp
from jax.experimental import pallas as pl

N = 10000
K = 8
E = N * K
HN = 64
HE = 64
NAT = 100
_BN = 1.0 / (1.0 + 1e-5) ** 0.5


def _ssp(x):
    return jax.nn.softplus(x) - jnp.log(2.0)


_BE = 1600


def _c3_body(g_ref, d_ref, m_ref, o_ref):
    d = d_ref[...]
    acc = jnp.zeros((_BE, HE), jnp.float32)
    for q in range(K):
        pre = (g_ref[:, q * 128:(q + 1) * 128] + d) * _BN
        f = pre[:, :HE]
        c = pre[:, HE:]
        acc += jax.nn.sigmoid(f) * jnp.tanh(c) * m_ref[:, q:q + 1]
    o_ref[...] = acc * _BN


def _c3_sum(G, D, mask):
    return pl.pallas_call(
        _c3_body,
        grid=(E // _BE,),
        in_specs=[
            pl.BlockSpec((_BE, K * 128), lambda i: (i, 0)),
            pl.BlockSpec((_BE, 128), lambda i: (i, 0)),
            pl.BlockSpec((_BE, K), lambda i: (i, 0)),
        ],
        out_specs=pl.BlockSpec((_BE, HE), lambda i: (i, 0)),
        out_shape=jax.ShapeDtypeStruct((E, HE), jnp.float32),
    )(G, D, mask)


def kernel(z, pos, params):
    sq = jnp.sum(pos * pos, axis=1)
    d2 = sq[:, None] + sq[None, :] - 2.0 * (pos @ pos.T)
    d2 = d2 + jnp.eye(N, dtype=pos.dtype) * 1e9
    _, nbr = jax.lax.top_k(-d2, K)
    row = nbr.reshape(-1)
    col = jnp.repeat(jnp.arange(N), K)

    rel = pos[col] - pos[row]
    dist = jnp.sqrt(jnp.sum(rel * rel, axis=-1))
    unit = rel / dist[:, None]

    W, b = params['emb0']
    h = _ssp(jax.nn.one_hot(z - 1, NAT, dtype=jnp.float32) @ W + b)
    W, b = params['emb1']
    h = _ssp(h @ W + b)
    W, b = params['emb2']
    node = h @ W + b

    offset = jnp.linspace(0.0, 5.0, HE)
    coeff = -0.5 / (offset[1] - offset[0]) ** 2
    edge = jnp.exp(coeff * (dist[:, None] - offset[None, :]) ** 2)

    mask = (col[:, None] != nbr[row]).astype(jnp.float32)

    for lp in params['layers']:
        W, b = lp['nu']
        pre = (jnp.repeat(node @ W[:HN], K, axis=0) + edge @ W[HN:] + b) * _BN
        gated = jax.nn.sigmoid(pre[:, :HN]) * jnp.tanh(pre[:, HN:])
        agg = gated.reshape(N, K, HN).sum(axis=1)
        node = jnp.tanh(node + agg * _BN)

        W, b = lp['c2']
        prod = jnp.repeat(node, K, axis=0) * node[row]
        c2 = (prod @ W + b) * _BN
        c2e = jax.nn.sigmoid(c2[:, :HE]) * jnp.tanh(c2[:, HE:]) * _BN

        W, b = lp['c3']
        Wi, Wj, Wk = W[:HN], W[HN:2 * HN], W[2 * HN:3 * HN]
        Wji, Wkj = W[3 * HN:3 * HN + HE], W[3 * HN + HE:]
        D = jnp.repeat(node @ Wi, K, axis=0) + edge @ Wji + b
        S = jnp.repeat(node @ Wj, K, axis=0) + (node @ Wk)[row] + edge @ Wkj
        G = S.reshape(N, K * 128)[row]
        c3e = _c3_sum(G, D, mask)

        edge = jnp.tanh(edge + c2e + c3e)

    W, b = params['fp0']
    h = _ssp(edge @ W + b)
    W, b = params['fp1']
    h = _ssp(h @ W + b)
    W, b = params['fp2']
    s = h @ W + b
    force = s * unit
    return force.reshape(N, K, 3).sum(axis=1)


# speedup vs baseline: 7.9665x; 1.6434x over previous
import functools

import jax
import jax.numpy as jnp
from jax.experimental import pallas as pl
from jax.experimental.pallas import tpu as pltpu

N = 10000
K = 8
E = N * K
HN = 64
HE = 64
NAT = 100
_BN = 1.0 / (1.0 + 1e-5) ** 0.5

_NPAD = 10240
_RB = 256
_NT = _NPAD // 128


def _knn_body(posr_ref, post_ref, sqc_ref, sqr_ref, out_ref, d2_ref):
    i = pl.program_id(0)
    dots = jnp.dot(posr_ref[...], post_ref[...],
                   preferred_element_type=jnp.float32)
    colid = jax.lax.broadcasted_iota(jnp.int32, (_RB, _NPAD), 1)
    rowid = i * _RB + jax.lax.broadcasted_iota(jnp.int32, (_RB, _NPAD), 0)
    d2 = (sqr_ref[...] + sqc_ref[...]) - 2.0 * dots
    d2_ref[...] = d2 + jnp.where(colid == rowid, 1e9, 0.0)

    lane = jax.lax.broadcasted_iota(jnp.int32, (_RB, 128), 1)
    big = jnp.float32(3e38)
    sels = []
    for k in range(K):
        def fold(t, carry):
            rv, ri = carry
            v = d2_ref[:, pl.ds(t * 128, 128)]
            gidx = t * 128 + lane
            for s in sels:
                v = jnp.where(gidx == s, big, v)
            m = v < rv
            return jnp.where(m, v, rv), jnp.where(m, gidx, ri)
        rv0 = jnp.full((_RB, 128), big, jnp.float32)
        ri0 = jnp.zeros((_RB, 128), jnp.int32)
        rv, ri = jax.lax.fori_loop(0, _NT, fold, (rv0, ri0))
        minv = jnp.min(rv, axis=1, keepdims=True)
        cand = jnp.where(rv == minv, ri, jnp.int32(2**31 - 1))
        sel = jnp.min(cand, axis=1, keepdims=True)
        sels.append(sel)
        out_ref[:, k:k + 1] = sel


def _knn(pos, sq):
    posr = jnp.zeros((_NPAD, 8), jnp.float32).at[:N, :3].set(pos)
    post = jnp.zeros((8, _NPAD), jnp.float32).at[:3, :N].set(pos.T)
    sqp = jnp.full((_NPAD,), 4e9, jnp.float32).at[:N].set(sq)
    nbr = pl.pallas_call(
        _knn_body,
        grid=(_NPAD // _RB,),
        in_specs=[
            pl.BlockSpec((_RB, 8), lambda i: (i, 0)),
            pl.BlockSpec((8, _NPAD), lambda i: (0, 0)),
            pl.BlockSpec((1, _NPAD), lambda i: (0, 0)),
            pl.BlockSpec((_RB, 1), lambda i: (i, 0)),
        ],
        out_specs=pl.BlockSpec((_RB, K), lambda i: (i, 0)),
        out_shape=jax.ShapeDtypeStruct((_NPAD, K), jnp.int32),
        scratch_shapes=[pltpu.VMEM((_RB, _NPAD), jnp.float32)],
    )(posr, post, sqp.reshape(1, _NPAD), sqp.reshape(_NPAD, 1))
    return nbr[:N]


def _ssp(x):
    return jax.nn.softplus(x) - jnp.log(2.0)


_BE = 1600


def _c3_body(g_ref, d_ref, m_ref, o_ref):
    d = d_ref[...]
    acc = jnp.zeros((_BE, HE), jnp.float32)
    for q in range(K):
        pre = (g_ref[:, q * 128:(q + 1) * 128] + d) * _BN
        f = pre[:, :HE]
        c = pre[:, HE:]
        acc += jax.nn.sigmoid(f) * jnp.tanh(c) * m_ref[:, q:q + 1]
    o_ref[...] = acc * _BN


def _c3_sum(G, D, mask):
    return pl.pallas_call(
        _c3_body,
        grid=(E // _BE,),
        in_specs=[
            pl.BlockSpec((_BE, K * 128), lambda i: (i, 0)),
            pl.BlockSpec((_BE, 128), lambda i: (i, 0)),
            pl.BlockSpec((_BE, K), lambda i: (i, 0)),
        ],
        out_specs=pl.BlockSpec((_BE, HE), lambda i: (i, 0)),
        out_shape=jax.ShapeDtypeStruct((E, HE), jnp.float32),
    )(G, D, mask)


def kernel(z, pos, params):
    sq = jnp.sum(pos * pos, axis=1)
    nbr = _knn(pos, sq)
    row = nbr.reshape(-1)
    col = jnp.repeat(jnp.arange(N), K)

    rel = pos[col] - pos[row]
    dist = jnp.sqrt(jnp.sum(rel * rel, axis=-1))
    unit = rel / dist[:, None]

    W, b = params['emb0']
    h = _ssp(jax.nn.one_hot(z - 1, NAT, dtype=jnp.float32) @ W + b)
    W, b = params['emb1']
    h = _ssp(h @ W + b)
    W, b = params['emb2']
    node = h @ W + b

    offset = jnp.linspace(0.0, 5.0, HE)
    coeff = -0.5 / (offset[1] - offset[0]) ** 2
    edge = jnp.exp(coeff * (dist[:, None] - offset[None, :]) ** 2)

    mask = (col[:, None] != nbr[row]).astype(jnp.float32)

    for lp in params['layers']:
        W, b = lp['nu']
        pre = (jnp.repeat(node @ W[:HN], K, axis=0) + edge @ W[HN:] + b) * _BN
        gated = jax.nn.sigmoid(pre[:, :HN]) * jnp.tanh(pre[:, HN:])
        agg = gated.reshape(N, K, HN).sum(axis=1)
        node = jnp.tanh(node + agg * _BN)

        W, b = lp['c2']
        prod = jnp.repeat(node, K, axis=0) * node[row]
        c2 = (prod @ W + b) * _BN
        c2e = jax.nn.sigmoid(c2[:, :HE]) * jnp.tanh(c2[:, HE:]) * _BN

        W, b = lp['c3']
        Wi, Wj, Wk = W[:HN], W[HN:2 * HN], W[2 * HN:3 * HN]
        Wji, Wkj = W[3 * HN:3 * HN + HE], W[3 * HN + HE:]
        D = jnp.repeat(node @ Wi, K, axis=0) + edge @ Wji + b
        S = jnp.repeat(node @ Wj, K, axis=0) + (node @ Wk)[row] + edge @ Wkj
        G = S.reshape(N, K * 128)[row]
        c3e = _c3_sum(G, D, mask)

        edge = jnp.tanh(edge + c2e + c3e)

    W, b = params['fp0']
    h = _ssp(edge @ W + b)
    W, b = params['fp1']
    h = _ssp(h @ W + b)
    W, b = params['fp2']
    s = h @ W + b
    force = s * unit
    return force.reshape(N, K, 3).sum(axis=1)


# speedup vs baseline: 8.1905x; 1.0281x over previous
import functools

import jax
import jax.numpy as jnp
from jax.experimental import pallas as pl
from jax.experimental.pallas import tpu as pltpu

N = 10000
K = 8
E = N * K
HN = 64
HE = 64
NAT = 100
_BN = 1.0 / (1.0 + 1e-5) ** 0.5

_NPAD = 10240
_RB = 64
_NT = _NPAD // 128


def _knn_body(posr_ref, post_ref, sqc_ref, sqr_ref, out_ref, d2_ref):
    i = pl.program_id(0)
    dots = jnp.dot(posr_ref[...], post_ref[...],
                   preferred_element_type=jnp.float32)
    colid = jax.lax.broadcasted_iota(jnp.int32, (_RB, _NPAD), 1)
    rowid = i * _RB + jax.lax.broadcasted_iota(jnp.int32, (_RB, _NPAD), 0)
    d2 = (sqr_ref[...] + sqc_ref[...]) - 2.0 * dots
    d2_ref[...] = d2 + jnp.where(colid == rowid, 1e9, 0.0)

    lane = jax.lax.broadcasted_iota(jnp.int32, (_RB, 128), 1)
    big = jnp.float32(3e38)
    imax = jnp.int32(2**31 - 1)
    pv = jnp.full((_RB, 1), -big, jnp.float32)
    pg = jnp.full((_RB, 1), -1, jnp.int32)
    for k in range(K):
        def fold(t, carry):
            rv, ri = carry
            v = d2_ref[:, pl.ds(t * 128, 128)]
            gidx = t * 128 + lane
            adm = (v > pv) | ((v == pv) & (gidx > pg))
            m = adm & (v < rv)
            return jnp.where(m, v, rv), jnp.where(m, gidx, ri)
        rv0 = jnp.full((_RB, 128), big, jnp.float32)
        ri0 = jnp.full((_RB, 128), imax, jnp.int32)
        rv, ri = jax.lax.fori_loop(0, _NT, fold, (rv0, ri0))
        minv = jnp.min(rv, axis=1, keepdims=True)
        cand = jnp.where(rv == minv, ri, imax)
        sel = jnp.min(cand, axis=1, keepdims=True)
        out_ref[:, k:k + 1] = sel
        pv, pg = minv, sel


def _knn(pos, sq):
    posr = jnp.zeros((_NPAD, 8), jnp.float32).at[:N, :3].set(pos)
    post = jnp.zeros((8, _NPAD), jnp.float32).at[:3, :N].set(pos.T)
    sqp = jnp.full((_NPAD,), 4e9, jnp.float32).at[:N].set(sq)
    nbr = pl.pallas_call(
        _knn_body,
        grid=(_NPAD // _RB,),
        in_specs=[
            pl.BlockSpec((_RB, 8), lambda i: (i, 0)),
            pl.BlockSpec((8, _NPAD), lambda i: (0, 0)),
            pl.BlockSpec((1, _NPAD), lambda i: (0, 0)),
            pl.BlockSpec((_RB, 1), lambda i: (i, 0)),
        ],
        out_specs=pl.BlockSpec((_RB, K), lambda i: (i, 0)),
        out_shape=jax.ShapeDtypeStruct((_NPAD, K), jnp.int32),
        scratch_shapes=[pltpu.VMEM((_RB, _NPAD), jnp.float32)],
    )(posr, post, sqp.reshape(1, _NPAD), sqp.reshape(_NPAD, 1))
    return nbr[:N]


def _ssp(x):
    return jax.nn.softplus(x) - jnp.log(2.0)


_BE = 1600


def _c3_body(g_ref, d_ref, m_ref, o_ref):
    d = d_ref[...]
    acc = jnp.zeros((_BE, HE), jnp.float32)
    for q in range(K):
        pre = (g_ref[:, q * 128:(q + 1) * 128] + d) * _BN
        f = pre[:, :HE]
        c = pre[:, HE:]
        acc += jax.nn.sigmoid(f) * jnp.tanh(c) * m_ref[:, q:q + 1]
    o_ref[...] = acc * _BN


def _c3_sum(G, D, mask):
    return pl.pallas_call(
        _c3_body,
        grid=(E // _BE,),
        in_specs=[
            pl.BlockSpec((_BE, K * 128), lambda i: (i, 0)),
            pl.BlockSpec((_BE, 128), lambda i: (i, 0)),
            pl.BlockSpec((_BE, K), lambda i: (i, 0)),
        ],
        out_specs=pl.BlockSpec((_BE, HE), lambda i: (i, 0)),
        out_shape=jax.ShapeDtypeStruct((E, HE), jnp.float32),
    )(G, D, mask)


def kernel(z, pos, params):
    sq = jnp.sum(pos * pos, axis=1)
    nbr = _knn(pos, sq)
    row = nbr.reshape(-1)
    col = jnp.repeat(jnp.arange(N), K)

    rel = pos[col] - pos[row]
    dist = jnp.sqrt(jnp.sum(rel * rel, axis=-1))
    unit = rel / dist[:, None]

    W, b = params['emb0']
    h = _ssp(jax.nn.one_hot(z - 1, NAT, dtype=jnp.float32) @ W + b)
    W, b = params['emb1']
    h = _ssp(h @ W + b)
    W, b = params['emb2']
    node = h @ W + b

    offset = jnp.linspace(0.0, 5.0, HE)
    coeff = -0.5 / (offset[1] - offset[0]) ** 2
    edge = jnp.exp(coeff * (dist[:, None] - offset[None, :]) ** 2)

    mask = (col[:, None] != nbr[row]).astype(jnp.float32)

    for lp in params['layers']:
        W, b = lp['nu']
        pre = (jnp.repeat(node @ W[:HN], K, axis=0) + edge @ W[HN:] + b) * _BN
        gated = jax.nn.sigmoid(pre[:, :HN]) * jnp.tanh(pre[:, HN:])
        agg = gated.reshape(N, K, HN).sum(axis=1)
        node = jnp.tanh(node + agg * _BN)

        W, b = lp['c2']
        prod = jnp.repeat(node, K, axis=0) * node[row]
        c2 = (prod @ W + b) * _BN
        c2e = jax.nn.sigmoid(c2[:, :HE]) * jnp.tanh(c2[:, HE:]) * _BN

        W, b = lp['c3']
        Wi, Wj, Wk = W[:HN], W[HN:2 * HN], W[2 * HN:3 * HN]
        Wji, Wkj = W[3 * HN:3 * HN + HE], W[3 * HN + HE:]
        D = jnp.repeat(node @ Wi, K, axis=0) + edge @ Wji + b
        S = jnp.repeat(node @ Wj, K, axis=0) + (node @ Wk)[row] + edge @ Wkj
        G = S.reshape(N, K * 128)[row]
        c3e = _c3_sum(G, D, mask)

        edge = jnp.tanh(edge + c2e + c3e)

    W, b = params['fp0']
    h = _ssp(edge @ W + b)
    W, b = params['fp1']
    h = _ssp(h @ W + b)
    W, b = params['fp2']
    s = h @ W + b
    force = s * unit
    return force.reshape(N, K, 3).sum(axis=1)


# speedup vs baseline: 9.7722x; 1.1931x over previous
import functools

import jax
import jax.numpy as jnp
from jax.experimental import pallas as pl
from jax.experimental.pallas import tpu as pltpu

N = 10000
K = 8
E = N * K
HN = 64
HE = 64
NAT = 100
_BN = 1.0 / (1.0 + 1e-5) ** 0.5

_NPAD = 10240
_RB = 64
_NT = _NPAD // 128


def _knn_body(posr_ref, post_ref, sqc_ref, sqr_ref, out_ref, d2_ref):
    i = pl.program_id(0)
    dots = jnp.dot(posr_ref[...], post_ref[...],
                   preferred_element_type=jnp.float32)
    colid = jax.lax.broadcasted_iota(jnp.int32, (_RB, _NPAD), 1)
    rowid = i * _RB + jax.lax.broadcasted_iota(jnp.int32, (_RB, _NPAD), 0)
    d2 = (sqr_ref[...] + sqc_ref[...]) - 2.0 * dots
    d2_ref[...] = d2 + jnp.where(colid == rowid, 1e9, 0.0)

    lane = jax.lax.broadcasted_iota(jnp.int32, (_RB, 128), 1)
    big = jnp.float32(3e38)
    imax = jnp.int32(2**31 - 1)
    _U = 8

    def lex_min(av, ai, bv, bi):
        m = (av < bv) | ((av == bv) & (ai < bi))
        return jnp.where(m, av, bv), jnp.where(m, ai, bi)

    pv = jnp.full((_RB, 1), -big, jnp.float32)
    pg = jnp.full((_RB, 1), -1, jnp.int32)
    for k in range(K // 2):
        def fold(t, carry):
            m1, i1, m2, i2 = carry
            for u in range(_U):
                v = d2_ref[:, pl.ds((t * _U + u) * 128, 128)]
                gidx = (t * _U + u) * 128 + lane
                adm = (v > pv) | ((v == pv) & (gidx > pg))
                v = jnp.where(adm, v, big)
                lt1 = v < m1
                lt2 = v < m2
                m2 = jnp.where(lt1, m1, jnp.where(lt2, v, m2))
                i2 = jnp.where(lt1, i1, jnp.where(lt2, gidx, i2))
                m1 = jnp.where(lt1, v, m1)
                i1 = jnp.where(lt1, gidx, i1)
            return m1, i1, m2, i2
        f0 = jnp.full((_RB, 128), big, jnp.float32)
        g0 = jnp.full((_RB, 128), imax, jnp.int32)
        m1, i1, m2, i2 = jax.lax.fori_loop(0, _NT // _U, fold,
                                           (f0, g0, f0 + 0.0, g0 + 0))
        v1 = jnp.min(m1, axis=1, keepdims=True)
        s1 = jnp.min(jnp.where(m1 == v1, i1, imax), axis=1, keepdims=True)
        hit = (m1 == v1) & (i1 == s1)
        r1 = jnp.where(hit, m2, m1)
        r1i = jnp.where(hit, i2, i1)
        v2 = jnp.min(r1, axis=1, keepdims=True)
        s2 = jnp.min(jnp.where(r1 == v2, r1i, imax), axis=1, keepdims=True)
        out_ref[:, 2 * k:2 * k + 1] = s1
        out_ref[:, 2 * k + 1:2 * k + 2] = s2
        pv, pg = v2, s2


def _knn(pos, sq):
    posr = jnp.zeros((_NPAD, 8), jnp.float32).at[:N, :3].set(pos)
    post = jnp.zeros((8, _NPAD), jnp.float32).at[:3, :N].set(pos.T)
    sqp = jnp.full((_NPAD,), 4e9, jnp.float32).at[:N].set(sq)
    nbr = pl.pallas_call(
        _knn_body,
        grid=(_NPAD // _RB,),
        in_specs=[
            pl.BlockSpec((_RB, 8), lambda i: (i, 0)),
            pl.BlockSpec((8, _NPAD), lambda i: (0, 0)),
            pl.BlockSpec((1, _NPAD), lambda i: (0, 0)),
            pl.BlockSpec((_RB, 1), lambda i: (i, 0)),
        ],
        out_specs=pl.BlockSpec((_RB, K), lambda i: (i, 0)),
        out_shape=jax.ShapeDtypeStruct((_NPAD, K), jnp.int32),
        scratch_shapes=[pltpu.VMEM((_RB, _NPAD), jnp.float32)],
    )(posr, post, sqp.reshape(1, _NPAD), sqp.reshape(_NPAD, 1))
    return nbr[:N]


def _ssp(x):
    return jax.nn.softplus(x) - jnp.log(2.0)


_BE = 1600


def _c3_body(g_ref, d_ref, m_ref, o_ref):
    d = d_ref[...]
    acc = jnp.zeros((_BE, HE), jnp.float32)
    for q in range(K):
        pre = (g_ref[:, q * 128:(q + 1) * 128] + d) * _BN
        f = pre[:, :HE]
        c = pre[:, HE:]
        acc += jax.nn.sigmoid(f) * jnp.tanh(c) * m_ref[:, q:q + 1]
    o_ref[...] = acc * _BN


def _c3_sum(G, D, mask):
    return pl.pallas_call(
        _c3_body,
        grid=(E // _BE,),
        in_specs=[
            pl.BlockSpec((_BE, K * 128), lambda i: (i, 0)),
            pl.BlockSpec((_BE, 128), lambda i: (i, 0)),
            pl.BlockSpec((_BE, K), lambda i: (i, 0)),
        ],
        out_specs=pl.BlockSpec((_BE, HE), lambda i: (i, 0)),
        out_shape=jax.ShapeDtypeStruct((E, HE), jnp.float32),
    )(G, D, mask)


def kernel(z, pos, params):
    sq = jnp.sum(pos * pos, axis=1)
    nbr = _knn(pos, sq)
    row = nbr.reshape(-1)
    col = jnp.repeat(jnp.arange(N), K)

    rel = pos[col] - pos[row]
    dist = jnp.sqrt(jnp.sum(rel * rel, axis=-1))
    unit = rel / dist[:, None]

    W, b = params['emb0']
    h = _ssp(jax.nn.one_hot(z - 1, NAT, dtype=jnp.float32) @ W + b)
    W, b = params['emb1']
    h = _ssp(h @ W + b)
    W, b = params['emb2']
    node = h @ W + b

    offset = jnp.linspace(0.0, 5.0, HE)
    coeff = -0.5 / (offset[1] - offset[0]) ** 2
    edge = jnp.exp(coeff * (dist[:, None] - offset[None, :]) ** 2)

    mask = (col[:, None] != nbr[row]).astype(jnp.float32)

    for lp in params['layers']:
        W, b = lp['nu']
        pre = (jnp.repeat(node @ W[:HN], K, axis=0) + edge @ W[HN:] + b) * _BN
        gated = jax.nn.sigmoid(pre[:, :HN]) * jnp.tanh(pre[:, HN:])
        agg = gated.reshape(N, K, HN).sum(axis=1)
        node = jnp.tanh(node + agg * _BN)

        W, b = lp['c2']
        prod = jnp.repeat(node, K, axis=0) * node[row]
        c2 = (prod @ W + b) * _BN
        c2e = jax.nn.sigmoid(c2[:, :HE]) * jnp.tanh(c2[:, HE:]) * _BN

        W, b = lp['c3']
        Wi, Wj, Wk = W[:HN], W[HN:2 * HN], W[2 * HN:3 * HN]
        Wji, Wkj = W[3 * HN:3 * HN + HE], W[3 * HN + HE:]
        D = jnp.repeat(node @ Wi, K, axis=0) + edge @ Wji + b
        S = jnp.repeat(node @ Wj, K, axis=0) + (node @ Wk)[row] + edge @ Wkj
        G = S.reshape(N, K * 128)[row]
        c3e = _c3_sum(G, D, mask)

        edge = jnp.tanh(edge + c2e + c3e)

    W, b = params['fp0']
    h = _ssp(edge @ W + b)
    W, b = params['fp1']
    h = _ssp(h @ W + b)
    W, b = params['fp2']
    s = h @ W + b
    force = s * unit
    return force.reshape(N, K, 3).sum(axis=1)


# speedup vs baseline: 9.8544x; 1.0084x over previous
import functools

import jax
import jax.numpy as jnp
from jax.experimental import pallas as pl
from jax.experimental.pallas import tpu as pltpu

N = 10000
K = 8
E = N * K
HN = 64
HE = 64
NAT = 100
_BN = 1.0 / (1.0 + 1e-5) ** 0.5

_NPAD = 10240
_RB = 64
_NT = _NPAD // 128


def _knn_body(posr_ref, post_ref, sqc_ref, sqr_ref, out_ref, d2_ref):
    i = pl.program_id(0)
    dots = jnp.dot(posr_ref[...], post_ref[...],
                   preferred_element_type=jnp.float32)
    colid = jax.lax.broadcasted_iota(jnp.int32, (_RB, _NPAD), 1)
    rowid = i * _RB + jax.lax.broadcasted_iota(jnp.int32, (_RB, _NPAD), 0)
    d2 = (sqr_ref[...] + sqc_ref[...]) - 2.0 * dots
    d2_ref[...] = d2 + jnp.where(colid == rowid, 1e9, 0.0)

    lane = jax.lax.broadcasted_iota(jnp.int32, (_RB, 128), 1)
    big = jnp.float32(3e38)
    imax = jnp.int32(2**31 - 1)
    _U = 8

    def lex_min(av, ai, bv, bi):
        m = (av < bv) | ((av == bv) & (ai < bi))
        return jnp.where(m, av, bv), jnp.where(m, ai, bi)

    pv = jnp.full((_RB, 1), -big, jnp.float32)
    pg = jnp.full((_RB, 1), -1, jnp.int32)
    for k in range(K // 2):
        def fold(t, carry):
            m1, i1, m2, i2 = carry
            for u in range(_U):
                v = d2_ref[:, pl.ds((t * _U + u) * 128, 128)]
                gidx = (t * _U + u) * 128 + lane
                adm = (v > pv) | ((v == pv) & (gidx > pg))
                v = jnp.where(adm, v, big)
                lt1 = v < m1
                lt2 = v < m2
                m2 = jnp.where(lt1, m1, jnp.where(lt2, v, m2))
                i2 = jnp.where(lt1, i1, jnp.where(lt2, gidx, i2))
                m1 = jnp.where(lt1, v, m1)
                i1 = jnp.where(lt1, gidx, i1)
            return m1, i1, m2, i2
        f0 = jnp.full((_RB, 128), big, jnp.float32)
        g0 = jnp.full((_RB, 128), imax, jnp.int32)
        m1, i1, m2, i2 = jax.lax.fori_loop(0, _NT // _U, fold,
                                           (f0, g0, f0 + 0.0, g0 + 0))
        v1 = jnp.min(m1, axis=1, keepdims=True)
        s1 = jnp.min(jnp.where(m1 == v1, i1, imax), axis=1, keepdims=True)
        hit = (m1 == v1) & (i1 == s1)
        r1 = jnp.where(hit, m2, m1)
        r1i = jnp.where(hit, i2, i1)
        v2 = jnp.min(r1, axis=1, keepdims=True)
        s2 = jnp.min(jnp.where(r1 == v2, r1i, imax), axis=1, keepdims=True)
        out_ref[:, 2 * k:2 * k + 1] = s1
        out_ref[:, 2 * k + 1:2 * k + 2] = s2
        pv, pg = v2, s2


def _knn(pos, sq):
    posr = jnp.zeros((_NPAD, 8), jnp.float32).at[:N, :3].set(pos)
    post = jnp.zeros((8, _NPAD), jnp.float32).at[:3, :N].set(pos.T)
    sqp = jnp.full((_NPAD,), 4e9, jnp.float32).at[:N].set(sq)
    nbr = pl.pallas_call(
        _knn_body,
        grid=(_NPAD // _RB,),
        in_specs=[
            pl.BlockSpec((_RB, 8), lambda i: (i, 0)),
            pl.BlockSpec((8, _NPAD), lambda i: (0, 0)),
            pl.BlockSpec((1, _NPAD), lambda i: (0, 0)),
            pl.BlockSpec((_RB, 1), lambda i: (i, 0)),
        ],
        out_specs=pl.BlockSpec((_RB, K), lambda i: (i, 0)),
        out_shape=jax.ShapeDtypeStruct((_NPAD, K), jnp.int32),
        scratch_shapes=[pltpu.VMEM((_RB, _NPAD), jnp.float32)],
    )(posr, post, sqp.reshape(1, _NPAD), sqp.reshape(_NPAD, 1))
    return nbr[:N]


def _ssp(x):
    return jax.nn.softplus(x) - jnp.log(2.0)


_BE = 1600


def _c3_body(g_ref, d_ref, m_ref, o_ref):
    d = d_ref[...]
    acc = jnp.zeros((_BE, HE), jnp.float32)
    for q in range(K):
        pre = (g_ref[:, q * 128:(q + 1) * 128].astype(jnp.float32) + d) * _BN
        f = pre[:, :HE]
        c = pre[:, HE:]
        acc += jax.nn.sigmoid(f) * jnp.tanh(c) * m_ref[:, q:q + 1]
    o_ref[...] = acc * _BN


def _c3_sum(G, D, mask):
    return pl.pallas_call(
        _c3_body,
        grid=(E // _BE,),
        in_specs=[
            pl.BlockSpec((_BE, K * 128), lambda i: (i, 0)),
            pl.BlockSpec((_BE, 128), lambda i: (i, 0)),
            pl.BlockSpec((_BE, K), lambda i: (i, 0)),
        ],
        out_specs=pl.BlockSpec((_BE, HE), lambda i: (i, 0)),
        out_shape=jax.ShapeDtypeStruct((E, HE), jnp.float32),
    )(G, D, mask)


def kernel(z, pos, params):
    sq = jnp.sum(pos * pos, axis=1)
    nbr = _knn(pos, sq)
    row = nbr.reshape(-1)
    col = jnp.repeat(jnp.arange(N), K)

    rel = pos[col] - pos[row]
    dist = jnp.sqrt(jnp.sum(rel * rel, axis=-1))
    unit = rel / dist[:, None]

    W, b = params['emb0']
    h = _ssp(jax.nn.one_hot(z - 1, NAT, dtype=jnp.float32) @ W + b)
    W, b = params['emb1']
    h = _ssp(h @ W + b)
    W, b = params['emb2']
    node = h @ W + b

    offset = jnp.linspace(0.0, 5.0, HE)
    coeff = -0.5 / (offset[1] - offset[0]) ** 2
    edge = jnp.exp(coeff * (dist[:, None] - offset[None, :]) ** 2)

    mask = (col[:, None] != nbr[row]).astype(jnp.float32)

    for lp in params['layers']:
        W, b = lp['nu']
        pre = (jnp.repeat(node @ W[:HN], K, axis=0) + edge @ W[HN:] + b) * _BN
        gated = jax.nn.sigmoid(pre[:, :HN]) * jnp.tanh(pre[:, HN:])
        agg = gated.reshape(N, K, HN).sum(axis=1)
        node = jnp.tanh(node + agg * _BN)

        W, b = lp['c2']
        prod = jnp.repeat(node, K, axis=0) * node[row]
        c2 = (prod @ W + b) * _BN
        c2e = jax.nn.sigmoid(c2[:, :HE]) * jnp.tanh(c2[:, HE:]) * _BN

        W, b = lp['c3']
        Wi, Wj, Wk = W[:HN], W[HN:2 * HN], W[2 * HN:3 * HN]
        Wji, Wkj = W[3 * HN:3 * HN + HE], W[3 * HN + HE:]
        D = jnp.repeat(node @ Wi, K, axis=0) + edge @ Wji + b
        S = jnp.repeat(node @ Wj, K, axis=0) + (node @ Wk)[row] + edge @ Wkj
        G = S.astype(jnp.bfloat16).reshape(N, K * 128)[row]
        c3e = _c3_sum(G, D, mask)

        edge = jnp.tanh(edge + c2e + c3e)

    W, b = params['fp0']
    h = _ssp(edge @ W + b)
    W, b = params['fp1']
    h = _ssp(h @ W + b)
    W, b = params['fp2']
    s = h @ W + b
    force = s * unit
    return force.reshape(N, K, 3).sum(axis=1)
